# baseline (speedup 1.0000x reference)
import jax
import jax.numpy as jnp
from jax import lax
from jax.experimental import pallas as pl
from jax.experimental.pallas import tpu as pltpu
from jax.experimental.pallas import tpu_sc as plsc

D = 768
DH = D // 2
KG = DH // 16
B = 128
L = 512
N = B * L
NC = 2
NS = 16
NW = NC * NS
NSLICE = 4
NTOK_S = N // NSLICE
TOK_W = NTOK_S // NW
CROWS = 16
NCHUNK = TOK_W // CROWS
NBUF = 4
EPS = 1e-12
BLK = 512
GRID_S = NTOK_S // BLK


def _gather_body(txt_ref, we_ref, out_ref, idx_v,
                 gb0, gb1, gb2, gb3, ob0, ob1, ob2, ob3,
                 gs0, gs1, gs2, gs3, os0, os1, os2, os3):
    c = lax.axis_index("c")
    s = lax.axis_index("s")
    wid = s * NC + c
    base = wid * TOK_W
    gbufs = [gb0, gb1, gb2, gb3]
    obufs = [ob0, ob1, ob2, ob3]
    gsems = [gs0, gs1, gs2, gs3]
    osems = [os0, os1, os2, os3]

    pltpu.sync_copy(txt_ref.at[pl.ds(base, TOK_W)], idx_v)

    def gather(ch, b):
        return pltpu.make_async_copy(
            we_ref.at[idx_v.at[pl.ds(ch * CROWS, CROWS)]], gbufs[b], gsems[b])

    def out_copy(ch, b):
        return pltpu.make_async_copy(
            obufs[b], out_ref.at[pl.ds(base + ch * CROWS, CROWS)], osems[b])

    half = jnp.full((16,), 0x8000, jnp.int32)
    lo_mask = jnp.full((16,), 0xFFFF, jnp.int32)
    hi_mask = jnp.full((16,), -0x10000, jnp.int32)

    def make_pack_row(gbuf, obuf):
        def pack_row(r, carry):
            for k in range(KG):
                a = gbuf[r, pl.ds(16 * k, 16)]
                bvec = gbuf[r, pl.ds(DH + 16 * k, 16)]
                ai = plsc.bitcast(a, jnp.int32)
                bi = plsc.bitcast(bvec, jnp.int32)
                ha = ((ai + half) >> 16) & lo_mask
                hb = (bi + half) & hi_mask
                obuf[r, pl.ds(16 * k, 16)] = ha | hb
            return carry
        return pack_row

    for b in range(NBUF - 1):
        gather(b, b).start()

    def outer(o, carry):
        for b in range(NBUF):
            ch = o * NBUF + b
            gather(ch, b).wait()
            if b == 0:
                @pl.when(o > 0)
                def _():
                    out_copy(ch - NBUF, b).wait()
            else:
                @pl.when(ch >= NBUF)
                def _():
                    out_copy(ch - NBUF, b).wait()
            lax.fori_loop(0, CROWS, make_pack_row(gbufs[b], obufs[b]), 0)
            out_copy(ch, b).start()
            chn = ch + NBUF - 1
            bn = (b + NBUF - 1) % NBUF

            @pl.when(chn < NCHUNK)
            def _():
                gather(chn, bn).start()
        return carry

    lax.fori_loop(0, NCHUNK // NBUF, outer, 0)

    for b in range(NBUF):
        out_copy(NCHUNK - NBUF + b, b).wait()


def _sc_gather(txt_slice, word_embeddings):
    mesh = plsc.VectorSubcoreMesh(core_axis_name="c", subcore_axis_name="s")
    return pl.kernel(
        _gather_body,
        out_type=jax.ShapeDtypeStruct((NTOK_S, DH), jnp.int32),
        mesh=mesh,
        compiler_params=pltpu.CompilerParams(needs_layout_passes=False),
        scratch_types=[pltpu.VMEM((TOK_W,), jnp.int32)]
        + [pltpu.VMEM((CROWS, D), jnp.float32)] * NBUF
        + [pltpu.VMEM((CROWS, DH), jnp.int32)] * NBUF
        + [pltpu.SemaphoreType.DMA] * (2 * NBUF),
    )(txt_slice, word_embeddings)


def _ln_first_body(we_ref, pe_ref, te_ref, g_ref, b_ref, o_ref):
    w = we_ref[...]
    lo = lax.bitcast_convert_type(w << 16, jnp.float32)
    hi = lax.bitcast_convert_type(w & jnp.int32(-0x10000), jnp.float32)
    x = jnp.concatenate([lo, hi], axis=-1) + pe_ref[...] + te_ref[...]
    mu = jnp.mean(x, axis=-1, keepdims=True)
    var = jnp.mean(x * x, axis=-1, keepdims=True) - mu * mu
    y = (x - mu) * lax.rsqrt(var + EPS)
    o_ref[...] = y * g_ref[...] + b_ref[...]


def _ln_chain_body(we_ref, pe_ref, te_ref, g_ref, b_ref, prev_ref, o_ref):
    _ln_first_body(we_ref, pe_ref, te_ref, g_ref, b_ref, o_ref)


def _tc_ln(sl, gathered, pe, te1, g2, b2, prev):
    in_specs = [
        pl.BlockSpec((BLK, DH), lambda i: (i, 0)),
        pl.BlockSpec((L, D), lambda i: (0, 0)),
        pl.BlockSpec((1, D), lambda i: (0, 0)),
        pl.BlockSpec((1, D), lambda i: (0, 0)),
        pl.BlockSpec((1, D), lambda i: (0, 0)),
    ]
    args = [gathered, pe, te1, g2, b2]
    body = _ln_first_body
    aliases = {}
    if prev is not None:
        in_specs.append(pl.BlockSpec((8, D), lambda i: (0, 0)))
        args.append(prev)
        body = _ln_chain_body
        aliases = {5: 0}
    base_blk = sl * GRID_S
    return pl.pallas_call(
        body,
        out_shape=jax.ShapeDtypeStruct((N, D), jnp.float32),
        grid=(GRID_S,),
        in_specs=in_specs,
        out_specs=pl.BlockSpec((BLK, D), lambda i, _b=base_blk: (i + _b, 0)),
        input_output_aliases=aliases,
        compiler_params=pltpu.CompilerParams(
            dimension_semantics=("arbitrary",)),
    )(*args)


@jax.jit
def _run(txt_flat, word_embeddings, position_embeddings,
         token_type_embeddings, ln_gamma, ln_beta):
    te1 = token_type_embeddings[:1]
    g2 = ln_gamma.reshape(1, D)
    b2 = ln_beta.reshape(1, D)
    gathered = [
        _sc_gather(lax.slice(txt_flat, (sl * NTOK_S,), ((sl + 1) * NTOK_S,)),
                   word_embeddings)
        for sl in range(NSLICE)
    ]
    out = None
    for sl in range(NSLICE):
        out = _tc_ln(sl, gathered[sl], position_embeddings, te1, g2, b2, out)
    return out


def kernel(txt, word_embeddings, position_embeddings, token_type_embeddings,
           ln_gamma, ln_beta):
    out = _run(txt.reshape(N), word_embeddings, position_embeddings,
               token_type_embeddings, ln_gamma, ln_beta)
    return out.reshape(B, L, D)


# speedup vs baseline: 1.2478x; 1.2478x over previous
import jax
import jax.numpy as jnp
from jax import lax
from jax.experimental import pallas as pl
from jax.experimental.pallas import tpu as pltpu
from jax.experimental.pallas import tpu_sc as plsc

D = 768
B = 128
L = 512
N = B * L
NC = 2
NS = 16
NW = NC * NS
NSLICE = 4
NTOK_S = N // NSLICE
TOK_W = NTOK_S // NW
CROWS = 32
NCHUNK = TOK_W // CROWS
NBUF = 4
EPS = 1e-12
BLK = 512
GRID_S = NTOK_S // BLK


def _gather_body(txt_ref, we_ref, out_ref, idx_v,
                 buf0, buf1, buf2, buf3,
                 gs0, gs1, gs2, gs3, os0, os1, os2, os3):
    c = lax.axis_index("c")
    s = lax.axis_index("s")
    wid = s * NC + c
    base = wid * TOK_W
    bufs = [buf0, buf1, buf2, buf3]
    gsems = [gs0, gs1, gs2, gs3]
    osems = [os0, os1, os2, os3]

    pltpu.sync_copy(txt_ref.at[pl.ds(base, TOK_W)], idx_v)

    def gather(ch, b):
        return pltpu.make_async_copy(
            we_ref.at[idx_v.at[pl.ds(ch * CROWS, CROWS)]], bufs[b], gsems[b])

    def out_copy(ch, b):
        return pltpu.make_async_copy(
            bufs[b], out_ref.at[pl.ds(base + ch * CROWS, CROWS)], osems[b])

    for b in range(NBUF - 1):
        gather(b, b).start()

    def outer(o, carry):
        for b in range(NBUF):
            ch = o * NBUF + b
            gather(ch, b).wait()
            out_copy(ch, b).start()
            chn = ch + NBUF - 1
            bn = (b + NBUF - 1) % NBUF

            def prefetch():
                out_copy(chn - NBUF, bn).wait()
                gather(chn, bn).start()

            if b == 0:
                @pl.when(o == 0)
                def _():
                    gather(chn, bn).start()

                @pl.when(jnp.logical_and(o > 0, chn < NCHUNK))
                def _():
                    prefetch()
            else:
                @pl.when(chn < NCHUNK)
                def _():
                    prefetch()
        return carry

    lax.fori_loop(0, NCHUNK // NBUF, outer, 0)

    for b in range(NBUF):
        out_copy(NCHUNK - NBUF + b, b).wait()


def _sc_gather(txt_slice, word_embeddings):
    mesh = plsc.VectorSubcoreMesh(core_axis_name="c", subcore_axis_name="s")
    return pl.kernel(
        _gather_body,
        out_type=jax.ShapeDtypeStruct((NTOK_S, D), jnp.float32),
        mesh=mesh,
        compiler_params=pltpu.CompilerParams(needs_layout_passes=False),
        scratch_types=[pltpu.VMEM((TOK_W,), jnp.int32)]
        + [pltpu.VMEM((CROWS, D), jnp.float32)] * NBUF
        + [pltpu.SemaphoreType.DMA] * (2 * NBUF),
    )(txt_slice, word_embeddings)


def _ln_first_body(we_ref, pe_ref, te_ref, g_ref, b_ref, o_ref):
    x = we_ref[...] + pe_ref[...] + te_ref[...]
    mu = jnp.mean(x, axis=-1, keepdims=True)
    var = jnp.mean(x * x, axis=-1, keepdims=True) - mu * mu
    y = (x - mu) * lax.rsqrt(var + EPS)
    o_ref[...] = y * g_ref[...] + b_ref[...]


def _ln_chain_body(we_ref, pe_ref, te_ref, g_ref, b_ref, prev_ref, o_ref):
    _ln_first_body(we_ref, pe_ref, te_ref, g_ref, b_ref, o_ref)


def _tc_ln(sl, gathered, pe, te1, g2, b2, prev):
    in_specs = [
        pl.BlockSpec((BLK, D), lambda i: (i, 0)),
        pl.BlockSpec((L, D), lambda i: (0, 0)),
        pl.BlockSpec((1, D), lambda i: (0, 0)),
        pl.BlockSpec((1, D), lambda i: (0, 0)),
        pl.BlockSpec((1, D), lambda i: (0, 0)),
    ]
    args = [gathered, pe, te1, g2, b2]
    body = _ln_first_body
    aliases = {}
    if prev is not None:
        in_specs.append(pl.BlockSpec((8, D), lambda i: (0, 0)))
        args.append(prev)
        body = _ln_chain_body
        aliases = {5: 0}
    base_blk = sl * GRID_S
    return pl.pallas_call(
        body,
        out_shape=jax.ShapeDtypeStruct((N, D), jnp.float32),
        grid=(GRID_S,),
        in_specs=in_specs,
        out_specs=pl.BlockSpec((BLK, D), lambda i, _b=base_blk: (i + _b, 0)),
        input_output_aliases=aliases,
        compiler_params=pltpu.CompilerParams(
            dimension_semantics=("arbitrary",)),
    )(*args)


@jax.jit
def _run(txt_flat, word_embeddings, position_embeddings,
         token_type_embeddings, ln_gamma, ln_beta):
    te1 = token_type_embeddings[:1]
    g2 = ln_gamma.reshape(1, D)
    b2 = ln_beta.reshape(1, D)
    gathered = [
        _sc_gather(lax.slice(txt_flat, (sl * NTOK_S,), ((sl + 1) * NTOK_S,)),
                   word_embeddings)
        for sl in range(NSLICE)
    ]
    out = None
    for sl in range(NSLICE):
        out = _tc_ln(sl, gathered[sl], position_embeddings, te1, g2, b2, out)
    return out


def kernel(txt, word_embeddings, position_embeddings, token_type_embeddings,
           ln_gamma, ln_beta):
    out = _run(txt.reshape(N), word_embeddings, position_embeddings,
               token_type_embeddings, ln_gamma, ln_beta)
    return out.reshape(B, L, D)
